# baseline (speedup 1.0000x reference)
import functools
from math import pi as _PI

import jax
import jax.numpy as jnp
from jax.experimental import pallas as pl
from jax.experimental.pallas import tpu as pltpu
from jax.experimental.pallas import tpu_sc as plsc

NQ = 4096
NC = 16384
K = 32
CUTOFF = 10.0
CIN = 256
NF = 128
NCLS = 32
NIND = 8

QB = 128
NBLK = NQ // QB
NE = NQ * K

_LOG2 = 0.6931471805599453
_RBF_STEP = CUTOFF / (NF - 1)
_RBF_COEFF = -0.5 / (_RBF_STEP * _RBF_STEP)

_SC_CORES = 2
_SC_SUBCORES = 16
_NW = _SC_CORES * _SC_SUBCORES
_EPW = NE // _NW
_CH = 128
_NCH = _EPW // _CH


def _ssp(x):
    return jnp.maximum(x, 0.0) + jnp.log(1.0 + jnp.exp(-jnp.abs(x))) - _LOG2


def _k0_body(attr_ref, w_ref, out_ref):
    out_ref[...] = jax.lax.dot_general(
        attr_ref[...], w_ref[...], (((1,), (1,)), ((), ())),
        preferred_element_type=jnp.float32)


def _precompute_a(node_attr_ctx, lin1_W):
    rb = 2048
    return pl.pallas_call(
        _k0_body,
        grid=(NC // rb,),
        in_specs=[
            pl.BlockSpec((rb, CIN), lambda i: (i, 0)),
            pl.BlockSpec((NF, CIN), lambda i: (0, 0)),
        ],
        out_specs=pl.BlockSpec((rb, NF), lambda i: (i, 0)),
        out_shape=jax.ShapeDtypeStruct((NC, NF), jnp.float32),
    )(node_attr_ctx, lin1_W)


TW = 512
NTILES = NC // TW
_BIGI = 1 << 30
_BIGF = 3e38


def _k1_body(t0_ref, t1_ref, pq_ref, pcT_ref, bq_ref, bc_ref,
             idx_ref, d2_ref, s_ref, sh_ref):
    i = pl.program_id(0)
    t0 = t0_ref[i]
    t1 = t1_ref[i]
    q = pq_ref[...]
    qsq = jnp.sum(q * q, axis=1, keepdims=True)
    bqv = bq_ref[...]

    tio = jax.lax.broadcasted_iota(jnp.int32, (QB, NTILES), 1)

    def fill(t, cm):
        lo = pl.multiple_of(t * TW, TW)
        pcTt = pcT_ref[:, pl.ds(lo, TW)]
        csq = jnp.sum(pcTt * pcTt, axis=0, keepdims=True)
        s = csq - 2.0 * jax.lax.dot_general(
            q, pcTt, (((1,), (0,)), ((), ())),
            preferred_element_type=jnp.float32)
        sh = csq - 2.0 * jax.lax.dot_general(
            q, pcTt, (((1,), (0,)), ((), ())),
            preferred_element_type=jnp.float32,
            precision=jax.lax.Precision.HIGHEST)
        bad = bqv != bc_ref[:, pl.ds(lo, TW)]
        s = jnp.where(bad, jnp.float32(1e30), s)
        s_ref[:, pl.ds(lo, TW)] = s
        sh_ref[:, pl.ds(lo, TW)] = jnp.where(bad, jnp.float32(1e30), sh)
        tmin = jnp.min(s, axis=1, keepdims=True)
        return jnp.where(tio == t, tmin, cm)

    cm0 = jax.lax.fori_loop(
        t0, t1, fill, jnp.full((QB, NTILES), _BIGF, jnp.float32))

    kio = jax.lax.broadcasted_iota(jnp.int32, (QB, K), 1)

    def kbody(k, carry):
        idxs, d2s, cm = carry
        m = jnp.min(cm, axis=1, keepdims=True)
        tsel = jnp.min(jnp.where(cm <= m, tio, jnp.int32(_BIGI)),
                       axis=1, keepdims=True)

        def scan(t, acc):
            am, d2p, cm = acc
            lo = pl.multiple_of(t * TW, TW)
            sv = s_ref[:, pl.ds(lo, TW)]
            iota = jax.lax.broadcasted_iota(jnp.int32, (QB, TW), 1) + lo
            pick = tsel == t
            cand = jnp.where(pick & (sv <= m), iota, jnp.int32(_BIGI))
            am_t = jnp.min(cand, axis=1, keepdims=True)
            hit = iota == am_t
            d2t = jnp.min(
                jnp.where(hit, sh_ref[:, pl.ds(lo, TW)], jnp.float32(_BIGF)),
                axis=1, keepdims=True)
            sv = jnp.where(hit, jnp.float32(2e30), sv)
            s_ref[:, pl.ds(lo, TW)] = sv
            tmin = jnp.min(sv, axis=1, keepdims=True)
            cm = jnp.where((tio == t) & pick, tmin, cm)
            return (jnp.minimum(am, am_t), jnp.minimum(d2p, d2t), cm)

        am, d2p, cm = jax.lax.fori_loop(
            t0, t1, scan,
            (jnp.full((QB, 1), _BIGI, jnp.int32),
             jnp.full((QB, 1), _BIGF, jnp.float32), cm))
        idxs = jnp.where(kio == k, am, idxs)
        d2s = jnp.where(kio == k, d2p + qsq, d2s)
        return idxs, d2s, cm

    idxs, d2s, _ = jax.lax.fori_loop(
        0, K, kbody,
        (jnp.zeros((QB, K), jnp.int32), jnp.zeros((QB, K), jnp.float32),
         cm0))
    idx_ref[...] = idxs
    d2_ref[...] = d2s


def _knn(pos_query8, pos_ctxT8, bq, bc, t0s, t1s):
    grid_spec = pltpu.PrefetchScalarGridSpec(
        num_scalar_prefetch=2,
        grid=(NBLK,),
        in_specs=[
            pl.BlockSpec((QB, 8), lambda i, *_: (i, 0)),
            pl.BlockSpec((8, NC), lambda i, *_: (0, 0)),
            pl.BlockSpec((QB, 1), lambda i, *_: (i, 0)),
            pl.BlockSpec((1, NC), lambda i, *_: (0, 0)),
        ],
        out_specs=[
            pl.BlockSpec((QB, K), lambda i, *_: (i, 0)),
            pl.BlockSpec((QB, K), lambda i, *_: (i, 0)),
        ],
        scratch_shapes=[pltpu.VMEM((QB, NC), jnp.float32),
                        pltpu.VMEM((QB, NC), jnp.float32)],
    )
    return pl.pallas_call(
        _k1_body,
        grid_spec=grid_spec,
        out_shape=[
            jax.ShapeDtypeStruct((NQ, K), jnp.int32),
            jax.ShapeDtypeStruct((NQ, K), jnp.float32),
        ],
    )(t0s, t1s, pos_query8, pos_ctxT8, bq, bc)


def _sc_gather_body(table, idxs, out, idx_v, buf, sem):
    wid = jax.lax.axis_index("s") * _SC_CORES + jax.lax.axis_index("c")
    base = wid * _EPW
    pltpu.sync_copy(idxs.at[pl.ds(base, _EPW)], idx_v)

    def body(c, _):
        pltpu.async_copy(table.at[idx_v.at[pl.ds(c * _CH, _CH)]],
                         buf, sem).wait()
        pltpu.sync_copy(buf, out.at[pl.ds(base + c * _CH, _CH)])
        return 0

    jax.lax.fori_loop(0, _NCH, body, 0)


@functools.lru_cache(maxsize=1)
def _sc_gather_fn():
    @functools.partial(
        pl.kernel,
        mesh=plsc.VectorSubcoreMesh(core_axis_name="c", subcore_axis_name="s"),
        out_type=jax.ShapeDtypeStruct((NE, NF), jnp.float32),
        scratch_types=[
            pltpu.VMEM((_EPW,), jnp.int32),
            pltpu.VMEM((_CH, NF), jnp.float32),
            pltpu.SemaphoreType.DMA,
        ],
    )
    def _sc_gather(table, idxs, out, idx_v, buf, sem):
        _sc_gather_body(table, idxs, out, idx_v, buf, sem)

    return _sc_gather


def _k3_body(g_ref, d2_ref,
             nn_W1, nn_b1, nn_W2, nn_b2, lin2_W, lin2_b,
             cls_W1, cls_b1, cls_W2, cls_b2,
             prp_W1, prp_b1, prp_W2, prp_b2,
             ycls_ref, yind_ref):
    d2 = jnp.maximum(d2_ref[...], 0.0)
    dist = jnp.sqrt(d2 + 1e-16)
    C = 0.5 * (jnp.cos(dist * jnp.float32(_PI / CUTOFF)) + 1.0)
    C = jnp.where(dist <= CUTOFF, C, 0.0)
    off3 = jax.lax.broadcasted_iota(
        jnp.int32, (1, 1, NF), 2).astype(jnp.float32) * jnp.float32(_RBF_STEP)
    arg = jnp.float32(_RBF_COEFF) * (dist.reshape(QB, K, 1) - off3) ** 2
    rbf = jnp.exp(arg).reshape(QB * K, NF)

    def lin(x, w, b=None):
        y = jax.lax.dot_general(x, w[...], (((1,), (1,)), ((), ())),
                                preferred_element_type=jnp.float32)
        if b is not None:
            y = y + b[...]
        return y

    W = lin(_ssp(lin(rbf, nn_W1, nn_b1)), nn_W2, nn_b2)
    h = lin(W * g_ref[...], lin2_W, lin2_b)

    hc = h.reshape(QB, K, NF) * C.reshape(QB, K, 1)
    y = jnp.sum(hc, axis=1)

    ycls_ref[...] = lin(_ssp(lin(y, cls_W1, cls_b1)), cls_W2, cls_b2)
    yind_ref[...] = lin(_ssp(lin(y, prp_W1, prp_b1)), prp_W2, prp_b2)


def _edge_mlp(G, d2e, p):
    EB = QB * K
    wspec = lambda shape: pl.BlockSpec(shape, lambda i: (0, 0))
    return pl.pallas_call(
        _k3_body,
        grid=(NBLK,),
        in_specs=[
            pl.BlockSpec((EB, NF), lambda i: (i, 0)),
            pl.BlockSpec((QB, K), lambda i: (i, 0)),
            wspec((NF, NF)), wspec((1, NF)),
            wspec((NF, NF)), wspec((1, NF)),
            wspec((NF, NF)), wspec((1, NF)),
            wspec((NF, NF)), wspec((1, NF)),
            wspec((NCLS, NF)), wspec((1, NCLS)),
            wspec((NF, NF)), wspec((1, NF)),
            wspec((NIND, NF)), wspec((1, NIND)),
        ],
        out_specs=[
            pl.BlockSpec((QB, NCLS), lambda i: (i, 0)),
            pl.BlockSpec((QB, NIND), lambda i: (i, 0)),
        ],
        out_shape=[
            jax.ShapeDtypeStruct((NQ, NCLS), jnp.float32),
            jax.ShapeDtypeStruct((NQ, NIND), jnp.float32),
        ],
    )(G, d2e,
      p['nn_W1'], p['nn_b1'].reshape(1, NF),
      p['nn_W2'], p['nn_b2'].reshape(1, NF),
      p['lin2_W'], p['lin2_b'].reshape(1, NF),
      p['cls_W1'], p['cls_b1'].reshape(1, NF),
      p['cls_W2'], p['cls_b2'].reshape(1, NCLS),
      p['prp_W1'], p['prp_b1'].reshape(1, NF),
      p['prp_W2'], p['prp_b2'].reshape(1, NIND))


def kernel(pos_query, pos_ctx, node_attr_ctx, batch_query, batch_ctx, params):
    pq8 = jnp.zeros((NQ, 8), jnp.float32).at[:, :3].set(pos_query)
    pcT8 = jnp.zeros((8, NC), jnp.float32).at[:3, :].set(pos_ctx.T)
    bq = batch_query.reshape(NQ, 1)
    bc = batch_ctx.reshape(1, NC)

    b_lo = batch_query.reshape(NBLK, QB)[:, 0]
    b_hi = batch_query.reshape(NBLK, QB)[:, -1]
    start = jnp.searchsorted(batch_ctx, b_lo, side='left').astype(jnp.int32)
    end = jnp.searchsorted(batch_ctx, b_hi, side='right').astype(jnp.int32)
    t0s = start // TW
    t1s = jnp.maximum((end + TW - 1) // TW, t0s + 1)

    A = _precompute_a(node_attr_ctx, params['lin1_W'])
    idx, d2 = _knn(pq8, pcT8, bq, bc, t0s, t1s)
    G = _sc_gather_fn()(A, idx.reshape(NE))
    y_cls, y_ind = _edge_mlp(G, d2, params)
    return (y_cls, y_ind)


# speedup vs baseline: 1.4472x; 1.4472x over previous
import functools
from math import pi as _PI

import jax
import jax.numpy as jnp
from jax.experimental import pallas as pl
from jax.experimental.pallas import tpu as pltpu
from jax.experimental.pallas import tpu_sc as plsc

NQ = 4096
NC = 16384
K = 32
CUTOFF = 10.0
CIN = 256
NF = 128
NCLS = 32
NIND = 8

QB = 128
NBLK = NQ // QB
NE = NQ * K

_LOG2 = 0.6931471805599453
_RBF_STEP = CUTOFF / (NF - 1)
_RBF_COEFF = -0.5 / (_RBF_STEP * _RBF_STEP)

_SC_CORES = 2
_SC_SUBCORES = 16
_NW = _SC_CORES * _SC_SUBCORES
_EPW = NE // _NW
_CH = 128
_NCH = _EPW // _CH


def _ssp(x):
    return jnp.maximum(x, 0.0) + jnp.log(1.0 + jnp.exp(-jnp.abs(x))) - _LOG2


def _k0_body(attr_ref, w_ref, out_ref):
    out_ref[...] = jax.lax.dot_general(
        attr_ref[...], w_ref[...], (((1,), (1,)), ((), ())),
        preferred_element_type=jnp.float32)


def _precompute_a(node_attr_ctx, lin1_W):
    rb = 2048
    return pl.pallas_call(
        _k0_body,
        grid=(NC // rb,),
        in_specs=[
            pl.BlockSpec((rb, CIN), lambda i: (i, 0)),
            pl.BlockSpec((NF, CIN), lambda i: (0, 0)),
        ],
        out_specs=pl.BlockSpec((rb, NF), lambda i: (i, 0)),
        out_shape=jax.ShapeDtypeStruct((NC, NF), jnp.float32),
    )(node_attr_ctx, lin1_W)


TW = 512
NTILES = NC // TW
CW = 4096
_BIGI = 1 << 30
_BIGF = 3e38


def _k1_body(t0_ref, t1_ref, pq_ref, pcT_ref, bq_ref, bc_ref,
             idx_ref, d2_ref, s_ref, sh_ref):
    i = pl.program_id(0)
    t0 = t0_ref[i]
    t1 = t1_ref[i]
    q = pq_ref[...]
    qsq = jnp.sum(q * q, axis=1, keepdims=True)
    bqv = bq_ref[...]

    tio = jax.lax.broadcasted_iota(jnp.int32, (QB, NTILES), 1)
    kio = jax.lax.broadcasted_iota(jnp.int32, (QB, K), 1)

    @pl.when(t1 - t0 <= CW // TW)
    def _compact():
        s_ref[:, pl.ds(0, CW)] = jnp.full((QB, CW), jnp.float32(_BIGF))

        def cfill(t, _):
            lo = pl.multiple_of(t * TW, TW)
            co = pl.multiple_of((t - t0) * TW, TW)
            pcTt = pcT_ref[:, pl.ds(lo, TW)]
            csq = jnp.sum(pcTt * pcTt, axis=0, keepdims=True)
            s = csq - 2.0 * jax.lax.dot_general(
                q, pcTt, (((1,), (0,)), ((), ())),
                preferred_element_type=jnp.float32)
            sh = csq - 2.0 * jax.lax.dot_general(
                q, pcTt, (((1,), (0,)), ((), ())),
                preferred_element_type=jnp.float32,
                precision=jax.lax.Precision.HIGHEST)
            bad = bqv != bc_ref[:, pl.ds(lo, TW)]
            s_ref[:, pl.ds(co, TW)] = jnp.where(bad, jnp.float32(1e30), s)
            sh_ref[:, pl.ds(co, TW)] = jnp.where(bad, jnp.float32(1e30), sh)
            return 0

        jax.lax.fori_loop(t0, t1, cfill, 0)
        base = t0 * TW
        iota = jax.lax.broadcasted_iota(jnp.int32, (QB, CW), 1) + base

        def ckbody(k, carry):
            idxs, d2s = carry
            sv = s_ref[:, pl.ds(0, CW)]
            m = jnp.min(sv, axis=1, keepdims=True)
            am = jnp.min(jnp.where(sv <= m, iota, jnp.int32(_BIGI)),
                         axis=1, keepdims=True)
            hit = iota == am
            d2k = jnp.min(
                jnp.where(hit, sh_ref[:, pl.ds(0, CW)], jnp.float32(_BIGF)),
                axis=1, keepdims=True)
            s_ref[:, pl.ds(0, CW)] = jnp.where(hit, jnp.float32(2e30), sv)
            return (jnp.where(kio == k, am, idxs),
                    jnp.where(kio == k, d2k + qsq, d2s))

        idxs, d2s = jax.lax.fori_loop(
            0, K, ckbody,
            (jnp.zeros((QB, K), jnp.int32), jnp.zeros((QB, K), jnp.float32)))
        idx_ref[...] = idxs
        d2_ref[...] = d2s

    @pl.when(t1 - t0 > CW // TW)
    def _tiled():
        _k1_tiled(t0, t1, q, qsq, bqv, tio, kio,
                  pcT_ref, bc_ref, idx_ref, d2_ref, s_ref, sh_ref)


def _k1_tiled(t0, t1, q, qsq, bqv, tio, kio,
              pcT_ref, bc_ref, idx_ref, d2_ref, s_ref, sh_ref):
    def fill(t, cm):
        lo = pl.multiple_of(t * TW, TW)
        pcTt = pcT_ref[:, pl.ds(lo, TW)]
        csq = jnp.sum(pcTt * pcTt, axis=0, keepdims=True)
        s = csq - 2.0 * jax.lax.dot_general(
            q, pcTt, (((1,), (0,)), ((), ())),
            preferred_element_type=jnp.float32)
        sh = csq - 2.0 * jax.lax.dot_general(
            q, pcTt, (((1,), (0,)), ((), ())),
            preferred_element_type=jnp.float32,
            precision=jax.lax.Precision.HIGHEST)
        bad = bqv != bc_ref[:, pl.ds(lo, TW)]
        s = jnp.where(bad, jnp.float32(1e30), s)
        s_ref[:, pl.ds(lo, TW)] = s
        sh_ref[:, pl.ds(lo, TW)] = jnp.where(bad, jnp.float32(1e30), sh)
        tmin = jnp.min(s, axis=1, keepdims=True)
        return jnp.where(tio == t, tmin, cm)

    cm0 = jax.lax.fori_loop(
        t0, t1, fill, jnp.full((QB, NTILES), _BIGF, jnp.float32))

    kio = jax.lax.broadcasted_iota(jnp.int32, (QB, K), 1)

    def kbody(k, carry):
        idxs, d2s, cm = carry
        m = jnp.min(cm, axis=1, keepdims=True)
        tsel = jnp.min(jnp.where(cm <= m, tio, jnp.int32(_BIGI)),
                       axis=1, keepdims=True)

        def scan(t, acc):
            am, d2p, cm = acc
            lo = pl.multiple_of(t * TW, TW)
            sv = s_ref[:, pl.ds(lo, TW)]
            iota = jax.lax.broadcasted_iota(jnp.int32, (QB, TW), 1) + lo
            pick = tsel == t
            cand = jnp.where(pick & (sv <= m), iota, jnp.int32(_BIGI))
            am_t = jnp.min(cand, axis=1, keepdims=True)
            hit = iota == am_t
            d2t = jnp.min(
                jnp.where(hit, sh_ref[:, pl.ds(lo, TW)], jnp.float32(_BIGF)),
                axis=1, keepdims=True)
            sv = jnp.where(hit, jnp.float32(2e30), sv)
            s_ref[:, pl.ds(lo, TW)] = sv
            tmin = jnp.min(sv, axis=1, keepdims=True)
            cm = jnp.where((tio == t) & pick, tmin, cm)
            return (jnp.minimum(am, am_t), jnp.minimum(d2p, d2t), cm)

        am, d2p, cm = jax.lax.fori_loop(
            t0, t1, scan,
            (jnp.full((QB, 1), _BIGI, jnp.int32),
             jnp.full((QB, 1), _BIGF, jnp.float32), cm))
        idxs = jnp.where(kio == k, am, idxs)
        d2s = jnp.where(kio == k, d2p + qsq, d2s)
        return idxs, d2s, cm

    idxs, d2s, _ = jax.lax.fori_loop(
        0, K, kbody,
        (jnp.zeros((QB, K), jnp.int32), jnp.zeros((QB, K), jnp.float32),
         cm0))
    idx_ref[...] = idxs
    d2_ref[...] = d2s


def _knn(pos_query8, pos_ctxT8, bq, bc, t0s, t1s):
    grid_spec = pltpu.PrefetchScalarGridSpec(
        num_scalar_prefetch=2,
        grid=(NBLK,),
        in_specs=[
            pl.BlockSpec((QB, 8), lambda i, *_: (i, 0)),
            pl.BlockSpec((8, NC), lambda i, *_: (0, 0)),
            pl.BlockSpec((QB, 1), lambda i, *_: (i, 0)),
            pl.BlockSpec((1, NC), lambda i, *_: (0, 0)),
        ],
        out_specs=[
            pl.BlockSpec((QB, K), lambda i, *_: (i, 0)),
            pl.BlockSpec((QB, K), lambda i, *_: (i, 0)),
        ],
        scratch_shapes=[pltpu.VMEM((QB, NC), jnp.float32),
                        pltpu.VMEM((QB, NC), jnp.float32)],
    )
    return pl.pallas_call(
        _k1_body,
        grid_spec=grid_spec,
        out_shape=[
            jax.ShapeDtypeStruct((NQ, K), jnp.int32),
            jax.ShapeDtypeStruct((NQ, K), jnp.float32),
        ],
    )(t0s, t1s, pos_query8, pos_ctxT8, bq, bc)


def _sc_gather_body(table, idxs, out, idx_v, buf, sem):
    wid = jax.lax.axis_index("s") * _SC_CORES + jax.lax.axis_index("c")
    base = wid * _EPW
    pltpu.sync_copy(idxs.at[pl.ds(base, _EPW)], idx_v)

    def body(c, _):
        pltpu.async_copy(table.at[idx_v.at[pl.ds(c * _CH, _CH)]],
                         buf, sem).wait()
        pltpu.sync_copy(buf, out.at[pl.ds(base + c * _CH, _CH)])
        return 0

    jax.lax.fori_loop(0, _NCH, body, 0)


@functools.lru_cache(maxsize=1)
def _sc_gather_fn():
    @functools.partial(
        pl.kernel,
        mesh=plsc.VectorSubcoreMesh(core_axis_name="c", subcore_axis_name="s"),
        out_type=jax.ShapeDtypeStruct((NE, NF), jnp.float32),
        scratch_types=[
            pltpu.VMEM((_EPW,), jnp.int32),
            pltpu.VMEM((_CH, NF), jnp.float32),
            pltpu.SemaphoreType.DMA,
        ],
    )
    def _sc_gather(table, idxs, out, idx_v, buf, sem):
        _sc_gather_body(table, idxs, out, idx_v, buf, sem)

    return _sc_gather


def _k3_body(g_ref, d2_ref,
             nn_W1, nn_b1, nn_W2, nn_b2, lin2_W, lin2_b,
             cls_W1, cls_b1, cls_W2, cls_b2,
             prp_W1, prp_b1, prp_W2, prp_b2,
             ycls_ref, yind_ref):
    d2 = jnp.maximum(d2_ref[...], 0.0)
    dist = jnp.sqrt(d2 + 1e-16)
    C = 0.5 * (jnp.cos(dist * jnp.float32(_PI / CUTOFF)) + 1.0)
    C = jnp.where(dist <= CUTOFF, C, 0.0)
    off3 = jax.lax.broadcasted_iota(
        jnp.int32, (1, 1, NF), 2).astype(jnp.float32) * jnp.float32(_RBF_STEP)
    arg = jnp.float32(_RBF_COEFF) * (dist.reshape(QB, K, 1) - off3) ** 2
    rbf = jnp.exp(arg).reshape(QB * K, NF)

    def lin(x, w, b=None):
        y = jax.lax.dot_general(x, w[...], (((1,), (1,)), ((), ())),
                                preferred_element_type=jnp.float32)
        if b is not None:
            y = y + b[...]
        return y

    W = lin(_ssp(lin(rbf, nn_W1, nn_b1)), nn_W2, nn_b2)
    h = lin(W * g_ref[...], lin2_W, lin2_b)

    hc = h.reshape(QB, K, NF) * C.reshape(QB, K, 1)
    y = jnp.sum(hc, axis=1)

    ycls_ref[...] = lin(_ssp(lin(y, cls_W1, cls_b1)), cls_W2, cls_b2)
    yind_ref[...] = lin(_ssp(lin(y, prp_W1, prp_b1)), prp_W2, prp_b2)


def _edge_mlp(G, d2e, p):
    EB = QB * K
    wspec = lambda shape: pl.BlockSpec(shape, lambda i: (0, 0))
    return pl.pallas_call(
        _k3_body,
        grid=(NBLK,),
        in_specs=[
            pl.BlockSpec((EB, NF), lambda i: (i, 0)),
            pl.BlockSpec((QB, K), lambda i: (i, 0)),
            wspec((NF, NF)), wspec((1, NF)),
            wspec((NF, NF)), wspec((1, NF)),
            wspec((NF, NF)), wspec((1, NF)),
            wspec((NF, NF)), wspec((1, NF)),
            wspec((NCLS, NF)), wspec((1, NCLS)),
            wspec((NF, NF)), wspec((1, NF)),
            wspec((NIND, NF)), wspec((1, NIND)),
        ],
        out_specs=[
            pl.BlockSpec((QB, NCLS), lambda i: (i, 0)),
            pl.BlockSpec((QB, NIND), lambda i: (i, 0)),
        ],
        out_shape=[
            jax.ShapeDtypeStruct((NQ, NCLS), jnp.float32),
            jax.ShapeDtypeStruct((NQ, NIND), jnp.float32),
        ],
    )(G, d2e,
      p['nn_W1'], p['nn_b1'].reshape(1, NF),
      p['nn_W2'], p['nn_b2'].reshape(1, NF),
      p['lin2_W'], p['lin2_b'].reshape(1, NF),
      p['cls_W1'], p['cls_b1'].reshape(1, NF),
      p['cls_W2'], p['cls_b2'].reshape(1, NCLS),
      p['prp_W1'], p['prp_b1'].reshape(1, NF),
      p['prp_W2'], p['prp_b2'].reshape(1, NIND))


def kernel(pos_query, pos_ctx, node_attr_ctx, batch_query, batch_ctx, params):
    pq8 = jnp.zeros((NQ, 8), jnp.float32).at[:, :3].set(pos_query)
    pcT8 = jnp.zeros((8, NC), jnp.float32).at[:3, :].set(pos_ctx.T)
    bq = batch_query.reshape(NQ, 1)
    bc = batch_ctx.reshape(1, NC)

    b_lo = batch_query.reshape(NBLK, QB)[:, 0]
    b_hi = batch_query.reshape(NBLK, QB)[:, -1]
    start = jnp.searchsorted(batch_ctx, b_lo, side='left').astype(jnp.int32)
    end = jnp.searchsorted(batch_ctx, b_hi, side='right').astype(jnp.int32)
    t0s = start // TW
    t1s = jnp.maximum((end + TW - 1) // TW, t0s + 1)

    A = _precompute_a(node_attr_ctx, params['lin1_W'])
    idx, d2 = _knn(pq8, pcT8, bq, bc, t0s, t1s)
    G = _sc_gather_fn()(A, idx.reshape(NE))
    y_cls, y_ind = _edge_mlp(G, d2, params)
    return (y_cls, y_ind)


# speedup vs baseline: 1.5163x; 1.0477x over previous
import functools
from math import pi as _PI

import jax
import jax.numpy as jnp
from jax.experimental import pallas as pl
from jax.experimental.pallas import tpu as pltpu
from jax.experimental.pallas import tpu_sc as plsc

NQ = 4096
NC = 16384
K = 32
CUTOFF = 10.0
CIN = 256
NF = 128
NCLS = 32
NIND = 8

QB = 128
NBLK = NQ // QB
NE = NQ * K

_LOG2 = 0.6931471805599453
_RBF_STEP = CUTOFF / (NF - 1)
_RBF_COEFF = -0.5 / (_RBF_STEP * _RBF_STEP)

_SC_CORES = 2
_SC_SUBCORES = 16
_NW = _SC_CORES * _SC_SUBCORES
_EPW = NE // _NW
_CH = 128
_NCH = _EPW // _CH


def _ssp(x):
    return jnp.maximum(x, 0.0) + jnp.log(1.0 + jnp.exp(-jnp.abs(x))) - _LOG2


def _k0_body(attr_ref, w_ref, out_ref):
    out_ref[...] = jax.lax.dot_general(
        attr_ref[...], w_ref[...], (((1,), (1,)), ((), ())),
        preferred_element_type=jnp.float32)


def _precompute_a(node_attr_ctx, lin1_W):
    rb = 2048
    return pl.pallas_call(
        _k0_body,
        grid=(NC // rb,),
        in_specs=[
            pl.BlockSpec((rb, CIN), lambda i: (i, 0)),
            pl.BlockSpec((NF, CIN), lambda i: (0, 0)),
        ],
        out_specs=pl.BlockSpec((rb, NF), lambda i: (i, 0)),
        out_shape=jax.ShapeDtypeStruct((NC, NF), jnp.float32),
    )(node_attr_ctx, lin1_W)


TW = 512
NTILES = NC // TW
CW = 4096
_BIGI = 1 << 30
_BIGF = 3e38


def _k1_body(t0_ref, t1_ref, pq_ref, pcT_ref, bq_ref, bc_ref,
             idx_ref, d2_ref, s_ref, sh_ref):
    i = pl.program_id(0)
    t0 = t0_ref[i]
    t1 = t1_ref[i]
    q = pq_ref[...]
    qsq = jnp.sum(q * q, axis=1, keepdims=True)
    bqv = bq_ref[...]

    tio = jax.lax.broadcasted_iota(jnp.int32, (QB, NTILES), 1)
    kio = jax.lax.broadcasted_iota(jnp.int32, (QB, K), 1)

    @pl.when(t1 - t0 <= CW // TW)
    def _compact():
        s_ref[:, pl.ds(0, CW)] = jnp.full((QB, CW), jnp.float32(_BIGF))

        def cfill(t, _):
            lo = pl.multiple_of(t * TW, TW)
            co = pl.multiple_of((t - t0) * TW, TW)
            pcTt = pcT_ref[:, pl.ds(lo, TW)]
            csq = jnp.sum(pcTt * pcTt, axis=0, keepdims=True)
            s = csq - 2.0 * jax.lax.dot_general(
                q, pcTt, (((1,), (0,)), ((), ())),
                preferred_element_type=jnp.float32)
            sh = csq - 2.0 * jax.lax.dot_general(
                q, pcTt, (((1,), (0,)), ((), ())),
                preferred_element_type=jnp.float32,
                precision=jax.lax.Precision.HIGHEST)
            bad = bqv != bc_ref[:, pl.ds(lo, TW)]
            s_ref[:, pl.ds(co, TW)] = jnp.where(bad, jnp.float32(1e30), s)
            sh_ref[:, pl.ds(co, TW)] = jnp.where(bad, jnp.float32(1e30), sh)
            return 0

        jax.lax.fori_loop(t0, t1, cfill, 0)
        base = t0 * TW
        iota = jax.lax.broadcasted_iota(jnp.int32, (QB, CW), 1) + base

        def ckbody(k2, carry):
            idxs, d2s = carry
            sv = s_ref[:, pl.ds(0, CW)]
            m1 = jnp.min(sv, axis=1, keepdims=True)
            am1 = jnp.min(jnp.where(sv <= m1, iota, jnp.int32(_BIGI)),
                          axis=1, keepdims=True)
            hit1 = iota == am1
            sv2 = jnp.where(hit1, jnp.float32(2e30), sv)
            m2 = jnp.min(sv2, axis=1, keepdims=True)
            am2 = jnp.min(jnp.where(sv2 <= m2, iota, jnp.int32(_BIGI)),
                          axis=1, keepdims=True)
            hit2 = iota == am2
            s_ref[:, pl.ds(0, CW)] = jnp.where(hit2, jnp.float32(2e30), sv2)
            shv = sh_ref[:, pl.ds(0, CW)]
            d2k1 = jnp.min(jnp.where(hit1, shv, jnp.float32(_BIGF)),
                           axis=1, keepdims=True)
            d2k2 = jnp.min(jnp.where(hit2, shv, jnp.float32(_BIGF)),
                           axis=1, keepdims=True)
            idxs = jnp.where(kio == 2 * k2, am1, idxs)
            idxs = jnp.where(kio == 2 * k2 + 1, am2, idxs)
            d2s = jnp.where(kio == 2 * k2, d2k1 + qsq, d2s)
            d2s = jnp.where(kio == 2 * k2 + 1, d2k2 + qsq, d2s)
            return idxs, d2s

        idxs, d2s = jax.lax.fori_loop(
            0, K // 2, ckbody,
            (jnp.zeros((QB, K), jnp.int32), jnp.zeros((QB, K), jnp.float32)))
        idx_ref[...] = idxs
        d2_ref[...] = d2s

    @pl.when(t1 - t0 > CW // TW)
    def _tiled():
        _k1_tiled(t0, t1, q, qsq, bqv, tio, kio,
                  pcT_ref, bc_ref, idx_ref, d2_ref, s_ref, sh_ref)


def _k1_tiled(t0, t1, q, qsq, bqv, tio, kio,
              pcT_ref, bc_ref, idx_ref, d2_ref, s_ref, sh_ref):
    def fill(t, cm):
        lo = pl.multiple_of(t * TW, TW)
        pcTt = pcT_ref[:, pl.ds(lo, TW)]
        csq = jnp.sum(pcTt * pcTt, axis=0, keepdims=True)
        s = csq - 2.0 * jax.lax.dot_general(
            q, pcTt, (((1,), (0,)), ((), ())),
            preferred_element_type=jnp.float32)
        sh = csq - 2.0 * jax.lax.dot_general(
            q, pcTt, (((1,), (0,)), ((), ())),
            preferred_element_type=jnp.float32,
            precision=jax.lax.Precision.HIGHEST)
        bad = bqv != bc_ref[:, pl.ds(lo, TW)]
        s = jnp.where(bad, jnp.float32(1e30), s)
        s_ref[:, pl.ds(lo, TW)] = s
        sh_ref[:, pl.ds(lo, TW)] = jnp.where(bad, jnp.float32(1e30), sh)
        tmin = jnp.min(s, axis=1, keepdims=True)
        return jnp.where(tio == t, tmin, cm)

    cm0 = jax.lax.fori_loop(
        t0, t1, fill, jnp.full((QB, NTILES), _BIGF, jnp.float32))

    kio = jax.lax.broadcasted_iota(jnp.int32, (QB, K), 1)

    def kbody(k, carry):
        idxs, d2s, cm = carry
        m = jnp.min(cm, axis=1, keepdims=True)
        tsel = jnp.min(jnp.where(cm <= m, tio, jnp.int32(_BIGI)),
                       axis=1, keepdims=True)

        def scan(t, acc):
            am, d2p, cm = acc
            lo = pl.multiple_of(t * TW, TW)
            sv = s_ref[:, pl.ds(lo, TW)]
            iota = jax.lax.broadcasted_iota(jnp.int32, (QB, TW), 1) + lo
            pick = tsel == t
            cand = jnp.where(pick & (sv <= m), iota, jnp.int32(_BIGI))
            am_t = jnp.min(cand, axis=1, keepdims=True)
            hit = iota == am_t
            d2t = jnp.min(
                jnp.where(hit, sh_ref[:, pl.ds(lo, TW)], jnp.float32(_BIGF)),
                axis=1, keepdims=True)
            sv = jnp.where(hit, jnp.float32(2e30), sv)
            s_ref[:, pl.ds(lo, TW)] = sv
            tmin = jnp.min(sv, axis=1, keepdims=True)
            cm = jnp.where((tio == t) & pick, tmin, cm)
            return (jnp.minimum(am, am_t), jnp.minimum(d2p, d2t), cm)

        am, d2p, cm = jax.lax.fori_loop(
            t0, t1, scan,
            (jnp.full((QB, 1), _BIGI, jnp.int32),
             jnp.full((QB, 1), _BIGF, jnp.float32), cm))
        idxs = jnp.where(kio == k, am, idxs)
        d2s = jnp.where(kio == k, d2p + qsq, d2s)
        return idxs, d2s, cm

    idxs, d2s, _ = jax.lax.fori_loop(
        0, K, kbody,
        (jnp.zeros((QB, K), jnp.int32), jnp.zeros((QB, K), jnp.float32),
         cm0))
    idx_ref[...] = idxs
    d2_ref[...] = d2s


def _knn(pos_query8, pos_ctxT8, bq, bc, t0s, t1s):
    grid_spec = pltpu.PrefetchScalarGridSpec(
        num_scalar_prefetch=2,
        grid=(NBLK,),
        in_specs=[
            pl.BlockSpec((QB, 8), lambda i, *_: (i, 0)),
            pl.BlockSpec((8, NC), lambda i, *_: (0, 0)),
            pl.BlockSpec((QB, 1), lambda i, *_: (i, 0)),
            pl.BlockSpec((1, NC), lambda i, *_: (0, 0)),
        ],
        out_specs=[
            pl.BlockSpec((QB, K), lambda i, *_: (i, 0)),
            pl.BlockSpec((QB, K), lambda i, *_: (i, 0)),
        ],
        scratch_shapes=[pltpu.VMEM((QB, NC), jnp.float32),
                        pltpu.VMEM((QB, NC), jnp.float32)],
    )
    return pl.pallas_call(
        _k1_body,
        grid_spec=grid_spec,
        out_shape=[
            jax.ShapeDtypeStruct((NQ, K), jnp.int32),
            jax.ShapeDtypeStruct((NQ, K), jnp.float32),
        ],
    )(t0s, t1s, pos_query8, pos_ctxT8, bq, bc)


def _sc_gather_body(table, idxs, out, idx_v, buf, sem):
    wid = jax.lax.axis_index("s") * _SC_CORES + jax.lax.axis_index("c")
    base = wid * _EPW
    pltpu.sync_copy(idxs.at[pl.ds(base, _EPW)], idx_v)

    def body(c, _):
        pltpu.async_copy(table.at[idx_v.at[pl.ds(c * _CH, _CH)]],
                         buf, sem).wait()
        pltpu.sync_copy(buf, out.at[pl.ds(base + c * _CH, _CH)])
        return 0

    jax.lax.fori_loop(0, _NCH, body, 0)


@functools.lru_cache(maxsize=1)
def _sc_gather_fn():
    @functools.partial(
        pl.kernel,
        mesh=plsc.VectorSubcoreMesh(core_axis_name="c", subcore_axis_name="s"),
        out_type=jax.ShapeDtypeStruct((NE, NF), jnp.float32),
        scratch_types=[
            pltpu.VMEM((_EPW,), jnp.int32),
            pltpu.VMEM((_CH, NF), jnp.float32),
            pltpu.SemaphoreType.DMA,
        ],
    )
    def _sc_gather(table, idxs, out, idx_v, buf, sem):
        _sc_gather_body(table, idxs, out, idx_v, buf, sem)

    return _sc_gather


def _k3_body(g_ref, d2_ref,
             nn_W1, nn_b1, nn_W2, nn_b2, lin2_W, lin2_b,
             cls_W1, cls_b1, cls_W2, cls_b2,
             prp_W1, prp_b1, prp_W2, prp_b2,
             ycls_ref, yind_ref):
    d2 = jnp.maximum(d2_ref[...], 0.0)
    dist = jnp.sqrt(d2 + 1e-16)
    C = 0.5 * (jnp.cos(dist * jnp.float32(_PI / CUTOFF)) + 1.0)
    C = jnp.where(dist <= CUTOFF, C, 0.0)
    off3 = jax.lax.broadcasted_iota(
        jnp.int32, (1, 1, NF), 2).astype(jnp.float32) * jnp.float32(_RBF_STEP)
    arg = jnp.float32(_RBF_COEFF) * (dist.reshape(QB, K, 1) - off3) ** 2
    rbf = jnp.exp(arg).reshape(QB * K, NF)

    def lin(x, w, b=None):
        y = jax.lax.dot_general(x, w[...], (((1,), (1,)), ((), ())),
                                preferred_element_type=jnp.float32)
        if b is not None:
            y = y + b[...]
        return y

    W = lin(_ssp(lin(rbf, nn_W1, nn_b1)), nn_W2, nn_b2)
    h = lin(W * g_ref[...], lin2_W, lin2_b)

    hc = h.reshape(QB, K, NF) * C.reshape(QB, K, 1)
    y = jnp.sum(hc, axis=1)

    ycls_ref[...] = lin(_ssp(lin(y, cls_W1, cls_b1)), cls_W2, cls_b2)
    yind_ref[...] = lin(_ssp(lin(y, prp_W1, prp_b1)), prp_W2, prp_b2)


def _edge_mlp(G, d2e, p):
    EB = QB * K
    wspec = lambda shape: pl.BlockSpec(shape, lambda i: (0, 0))
    return pl.pallas_call(
        _k3_body,
        grid=(NBLK,),
        in_specs=[
            pl.BlockSpec((EB, NF), lambda i: (i, 0)),
            pl.BlockSpec((QB, K), lambda i: (i, 0)),
            wspec((NF, NF)), wspec((1, NF)),
            wspec((NF, NF)), wspec((1, NF)),
            wspec((NF, NF)), wspec((1, NF)),
            wspec((NF, NF)), wspec((1, NF)),
            wspec((NCLS, NF)), wspec((1, NCLS)),
            wspec((NF, NF)), wspec((1, NF)),
            wspec((NIND, NF)), wspec((1, NIND)),
        ],
        out_specs=[
            pl.BlockSpec((QB, NCLS), lambda i: (i, 0)),
            pl.BlockSpec((QB, NIND), lambda i: (i, 0)),
        ],
        out_shape=[
            jax.ShapeDtypeStruct((NQ, NCLS), jnp.float32),
            jax.ShapeDtypeStruct((NQ, NIND), jnp.float32),
        ],
    )(G, d2e,
      p['nn_W1'], p['nn_b1'].reshape(1, NF),
      p['nn_W2'], p['nn_b2'].reshape(1, NF),
      p['lin2_W'], p['lin2_b'].reshape(1, NF),
      p['cls_W1'], p['cls_b1'].reshape(1, NF),
      p['cls_W2'], p['cls_b2'].reshape(1, NCLS),
      p['prp_W1'], p['prp_b1'].reshape(1, NF),
      p['prp_W2'], p['prp_b2'].reshape(1, NIND))


def kernel(pos_query, pos_ctx, node_attr_ctx, batch_query, batch_ctx, params):
    pq8 = jnp.zeros((NQ, 8), jnp.float32).at[:, :3].set(pos_query)
    pcT8 = jnp.zeros((8, NC), jnp.float32).at[:3, :].set(pos_ctx.T)
    bq = batch_query.reshape(NQ, 1)
    bc = batch_ctx.reshape(1, NC)

    b_lo = batch_query.reshape(NBLK, QB)[:, 0]
    b_hi = batch_query.reshape(NBLK, QB)[:, -1]
    start = jnp.searchsorted(batch_ctx, b_lo, side='left').astype(jnp.int32)
    end = jnp.searchsorted(batch_ctx, b_hi, side='right').astype(jnp.int32)
    t0s = start // TW
    t1s = jnp.maximum((end + TW - 1) // TW, t0s + 1)

    A = _precompute_a(node_attr_ctx, params['lin1_W'])
    idx, d2 = _knn(pq8, pcT8, bq, bc, t0s, t1s)
    G = _sc_gather_fn()(A, idx.reshape(NE))
    y_cls, y_ind = _edge_mlp(G, d2, params)
    return (y_cls, y_ind)


# speedup vs baseline: 2.2319x; 1.4720x over previous
import functools
from math import pi as _PI

import jax
import jax.numpy as jnp
from jax.experimental import pallas as pl
from jax.experimental.pallas import tpu as pltpu
from jax.experimental.pallas import tpu_sc as plsc

NQ = 4096
NC = 16384
K = 32
CUTOFF = 10.0
CIN = 256
NF = 128
NCLS = 32
NIND = 8

QB = 128
NBLK = NQ // QB
NE = NQ * K

_LOG2 = 0.6931471805599453
_RBF_STEP = CUTOFF / (NF - 1)
_RBF_COEFF = -0.5 / (_RBF_STEP * _RBF_STEP)

_SC_CORES = 2
_SC_SUBCORES = 16
_NW = _SC_CORES * _SC_SUBCORES
_EPW = NE // _NW
_CH = 128
_NCH = _EPW // _CH


def _ssp(x):
    return jnp.maximum(x, 0.0) + jnp.log(1.0 + jnp.exp(-jnp.abs(x))) - _LOG2


def _k0_body(attr_ref, w_ref, out_ref):
    out_ref[...] = jax.lax.dot_general(
        attr_ref[...], w_ref[...], (((1,), (1,)), ((), ())),
        preferred_element_type=jnp.float32)


def _precompute_a(node_attr_ctx, lin1_W):
    rb = 2048
    return pl.pallas_call(
        _k0_body,
        grid=(NC // rb,),
        in_specs=[
            pl.BlockSpec((rb, CIN), lambda i: (i, 0)),
            pl.BlockSpec((NF, CIN), lambda i: (0, 0)),
        ],
        out_specs=pl.BlockSpec((rb, NF), lambda i: (i, 0)),
        out_shape=jax.ShapeDtypeStruct((NC, NF), jnp.float32),
    )(node_attr_ctx, lin1_W)


TW = 512
NTILES = NC // TW
CW1 = 3072
CW2 = 5632
_BIGI = 1 << 30
_BIGF = 3e38


def _k1_body(t0_ref, t1_ref, pq_ref, pcT_ref, bq_ref, bc_ref,
             idx_ref, d2_ref, s_ref, sh_ref):
    i = pl.program_id(0)
    t0 = t0_ref[i]
    t1 = t1_ref[i]
    q = pq_ref[...]
    qsq = jnp.sum(q * q, axis=1, keepdims=True)
    bqv = bq_ref[...]

    tio = jax.lax.broadcasted_iota(jnp.int32, (QB, NTILES), 1)
    kio = jax.lax.broadcasted_iota(jnp.int32, (QB, K), 1)

    def _compact(cw):
        s_ref[:, pl.ds(0, cw)] = jnp.full((QB, cw), jnp.float32(_BIGF))

        def cfill(t, _):
            lo = pl.multiple_of(t * TW, TW)
            co = pl.multiple_of((t - t0) * TW, TW)
            pcTt = pcT_ref[:, pl.ds(lo, TW)]
            csq = jnp.sum(pcTt * pcTt, axis=0, keepdims=True)
            s = csq - 2.0 * jax.lax.dot_general(
                q, pcTt, (((1,), (0,)), ((), ())),
                preferred_element_type=jnp.float32)
            sh = csq - 2.0 * jax.lax.dot_general(
                q, pcTt, (((1,), (0,)), ((), ())),
                preferred_element_type=jnp.float32,
                precision=jax.lax.Precision.HIGHEST)
            bad = bqv != bc_ref[:, pl.ds(lo, TW)]
            s_ref[:, pl.ds(co, TW)] = jnp.where(bad, jnp.float32(1e30), s)
            sh_ref[:, pl.ds(co, TW)] = jnp.where(bad, jnp.float32(1e30), sh)
            return 0

        jax.lax.fori_loop(t0, t1, cfill, 0)
        base = t0 * TW
        iota = jax.lax.broadcasted_iota(jnp.int32, (QB, cw), 1) + base

        def ckbody(k2, carry):
            idxs, d2s = carry
            sv = s_ref[:, pl.ds(0, cw)]
            m1 = jnp.min(sv, axis=1, keepdims=True)
            am1 = jnp.min(jnp.where(sv <= m1, iota, jnp.int32(_BIGI)),
                          axis=1, keepdims=True)
            hit1 = iota == am1
            sv2 = jnp.where(hit1, jnp.float32(2e30), sv)
            m2 = jnp.min(sv2, axis=1, keepdims=True)
            am2 = jnp.min(jnp.where(sv2 <= m2, iota, jnp.int32(_BIGI)),
                          axis=1, keepdims=True)
            hit2 = iota == am2
            s_ref[:, pl.ds(0, cw)] = jnp.where(hit2, jnp.float32(2e30), sv2)
            shv = sh_ref[:, pl.ds(0, cw)]
            d2k1 = jnp.min(jnp.where(hit1, shv, jnp.float32(_BIGF)),
                           axis=1, keepdims=True)
            d2k2 = jnp.min(jnp.where(hit2, shv, jnp.float32(_BIGF)),
                           axis=1, keepdims=True)
            idxs = jnp.where(kio == 2 * k2, am1, idxs)
            idxs = jnp.where(kio == 2 * k2 + 1, am2, idxs)
            d2s = jnp.where(kio == 2 * k2, d2k1 + qsq, d2s)
            d2s = jnp.where(kio == 2 * k2 + 1, d2k2 + qsq, d2s)
            return idxs, d2s

        idxs, d2s = jax.lax.fori_loop(
            0, K // 2, ckbody,
            (jnp.zeros((QB, K), jnp.int32), jnp.zeros((QB, K), jnp.float32)))
        idx_ref[...] = idxs
        d2_ref[...] = d2s

    nt = t1 - t0
    pl.when(nt <= CW1 // TW)(lambda: _compact(CW1))
    pl.when((nt > CW1 // TW) & (nt <= CW2 // TW))(lambda: _compact(CW2))

    @pl.when(nt > CW2 // TW)
    def _tiled():
        _k1_tiled(t0, t1, q, qsq, bqv, tio, kio,
                  pcT_ref, bc_ref, idx_ref, d2_ref, s_ref, sh_ref)


def _k1_tiled(t0, t1, q, qsq, bqv, tio, kio,
              pcT_ref, bc_ref, idx_ref, d2_ref, s_ref, sh_ref):
    def fill(t, cm):
        lo = pl.multiple_of(t * TW, TW)
        pcTt = pcT_ref[:, pl.ds(lo, TW)]
        csq = jnp.sum(pcTt * pcTt, axis=0, keepdims=True)
        s = csq - 2.0 * jax.lax.dot_general(
            q, pcTt, (((1,), (0,)), ((), ())),
            preferred_element_type=jnp.float32)
        sh = csq - 2.0 * jax.lax.dot_general(
            q, pcTt, (((1,), (0,)), ((), ())),
            preferred_element_type=jnp.float32,
            precision=jax.lax.Precision.HIGHEST)
        bad = bqv != bc_ref[:, pl.ds(lo, TW)]
        s = jnp.where(bad, jnp.float32(1e30), s)
        s_ref[:, pl.ds(lo, TW)] = s
        sh_ref[:, pl.ds(lo, TW)] = jnp.where(bad, jnp.float32(1e30), sh)
        tmin = jnp.min(s, axis=1, keepdims=True)
        return jnp.where(tio == t, tmin, cm)

    cm0 = jax.lax.fori_loop(
        t0, t1, fill, jnp.full((QB, NTILES), _BIGF, jnp.float32))

    kio = jax.lax.broadcasted_iota(jnp.int32, (QB, K), 1)

    def kbody(k, carry):
        idxs, d2s, cm = carry
        m = jnp.min(cm, axis=1, keepdims=True)
        tsel = jnp.min(jnp.where(cm <= m, tio, jnp.int32(_BIGI)),
                       axis=1, keepdims=True)

        def scan(t, acc):
            am, d2p, cm = acc
            lo = pl.multiple_of(t * TW, TW)
            sv = s_ref[:, pl.ds(lo, TW)]
            iota = jax.lax.broadcasted_iota(jnp.int32, (QB, TW), 1) + lo
            pick = tsel == t
            cand = jnp.where(pick & (sv <= m), iota, jnp.int32(_BIGI))
            am_t = jnp.min(cand, axis=1, keepdims=True)
            hit = iota == am_t
            d2t = jnp.min(
                jnp.where(hit, sh_ref[:, pl.ds(lo, TW)], jnp.float32(_BIGF)),
                axis=1, keepdims=True)
            sv = jnp.where(hit, jnp.float32(2e30), sv)
            s_ref[:, pl.ds(lo, TW)] = sv
            tmin = jnp.min(sv, axis=1, keepdims=True)
            cm = jnp.where((tio == t) & pick, tmin, cm)
            return (jnp.minimum(am, am_t), jnp.minimum(d2p, d2t), cm)

        am, d2p, cm = jax.lax.fori_loop(
            t0, t1, scan,
            (jnp.full((QB, 1), _BIGI, jnp.int32),
             jnp.full((QB, 1), _BIGF, jnp.float32), cm))
        idxs = jnp.where(kio == k, am, idxs)
        d2s = jnp.where(kio == k, d2p + qsq, d2s)
        return idxs, d2s, cm

    idxs, d2s, _ = jax.lax.fori_loop(
        0, K, kbody,
        (jnp.zeros((QB, K), jnp.int32), jnp.zeros((QB, K), jnp.float32),
         cm0))
    idx_ref[...] = idxs
    d2_ref[...] = d2s


def _knn(pos_query8, pos_ctxT8, bq, bc, t0s, t1s):
    grid_spec = pltpu.PrefetchScalarGridSpec(
        num_scalar_prefetch=2,
        grid=(NBLK,),
        in_specs=[
            pl.BlockSpec((QB, 8), lambda i, *_: (i, 0)),
            pl.BlockSpec((8, NC), lambda i, *_: (0, 0)),
            pl.BlockSpec((QB, 1), lambda i, *_: (i, 0)),
            pl.BlockSpec((1, NC), lambda i, *_: (0, 0)),
        ],
        out_specs=[
            pl.BlockSpec((QB, K), lambda i, *_: (i, 0)),
            pl.BlockSpec((QB, K), lambda i, *_: (i, 0)),
        ],
        scratch_shapes=[pltpu.VMEM((QB, NC), jnp.float32),
                        pltpu.VMEM((QB, NC), jnp.float32)],
    )
    return pl.pallas_call(
        _k1_body,
        grid_spec=grid_spec,
        out_shape=[
            jax.ShapeDtypeStruct((NQ, K), jnp.int32),
            jax.ShapeDtypeStruct((NQ, K), jnp.float32),
        ],
    )(t0s, t1s, pos_query8, pos_ctxT8, bq, bc)


def _sc_gather_body(table, idxs, out, idx_v, buf, sem):
    wid = jax.lax.axis_index("s") * _SC_CORES + jax.lax.axis_index("c")
    base = wid * _EPW
    pltpu.sync_copy(idxs.at[pl.ds(base, _EPW)], idx_v)

    def body(c, _):
        pltpu.async_copy(table.at[idx_v.at[pl.ds(c * _CH, _CH)]],
                         buf, sem).wait()
        pltpu.sync_copy(buf, out.at[pl.ds(base + c * _CH, _CH)])
        return 0

    jax.lax.fori_loop(0, _NCH, body, 0)


@functools.lru_cache(maxsize=1)
def _sc_gather_fn():
    @functools.partial(
        pl.kernel,
        mesh=plsc.VectorSubcoreMesh(core_axis_name="c", subcore_axis_name="s"),
        out_type=jax.ShapeDtypeStruct((NE, NF), jnp.float32),
        scratch_types=[
            pltpu.VMEM((_EPW,), jnp.int32),
            pltpu.VMEM((_CH, NF), jnp.float32),
            pltpu.SemaphoreType.DMA,
        ],
    )
    def _sc_gather(table, idxs, out, idx_v, buf, sem):
        _sc_gather_body(table, idxs, out, idx_v, buf, sem)

    return _sc_gather


def _k3_body(g_ref, d2_ref,
             nn_W1, nn_b1, nn_W2, nn_b2, lin2_W, lin2_b,
             cls_W1, cls_b1, cls_W2, cls_b2,
             prp_W1, prp_b1, prp_W2, prp_b2,
             ycls_ref, yind_ref):
    d2 = jnp.maximum(d2_ref[...], 0.0)
    dist = jnp.sqrt(d2 + 1e-16)
    C = 0.5 * (jnp.cos(dist * jnp.float32(_PI / CUTOFF)) + 1.0)
    C = jnp.where(dist <= CUTOFF, C, 0.0)
    off3 = jax.lax.broadcasted_iota(
        jnp.int32, (1, 1, NF), 2).astype(jnp.float32) * jnp.float32(_RBF_STEP)
    arg = jnp.float32(_RBF_COEFF) * (dist.reshape(QB, K, 1) - off3) ** 2
    rbf = jnp.exp(arg).reshape(QB * K, NF)

    def lin(x, w, b=None):
        y = jax.lax.dot_general(x, w[...], (((1,), (1,)), ((), ())),
                                preferred_element_type=jnp.float32)
        if b is not None:
            y = y + b[...]
        return y

    W = lin(_ssp(lin(rbf, nn_W1, nn_b1)), nn_W2, nn_b2)
    h = lin(W * g_ref[...], lin2_W, lin2_b)

    hc = h.reshape(QB, K, NF) * C.reshape(QB, K, 1)
    y = jnp.sum(hc, axis=1)

    ycls_ref[...] = lin(_ssp(lin(y, cls_W1, cls_b1)), cls_W2, cls_b2)
    yind_ref[...] = lin(_ssp(lin(y, prp_W1, prp_b1)), prp_W2, prp_b2)


def _edge_mlp(G, d2e, p):
    EB = QB * K
    wspec = lambda shape: pl.BlockSpec(shape, lambda i: (0, 0))
    return pl.pallas_call(
        _k3_body,
        grid=(NBLK,),
        in_specs=[
            pl.BlockSpec((EB, NF), lambda i: (i, 0)),
            pl.BlockSpec((QB, K), lambda i: (i, 0)),
            wspec((NF, NF)), wspec((1, NF)),
            wspec((NF, NF)), wspec((1, NF)),
            wspec((NF, NF)), wspec((1, NF)),
            wspec((NF, NF)), wspec((1, NF)),
            wspec((NCLS, NF)), wspec((1, NCLS)),
            wspec((NF, NF)), wspec((1, NF)),
            wspec((NIND, NF)), wspec((1, NIND)),
        ],
        out_specs=[
            pl.BlockSpec((QB, NCLS), lambda i: (i, 0)),
            pl.BlockSpec((QB, NIND), lambda i: (i, 0)),
        ],
        out_shape=[
            jax.ShapeDtypeStruct((NQ, NCLS), jnp.float32),
            jax.ShapeDtypeStruct((NQ, NIND), jnp.float32),
        ],
    )(G, d2e,
      p['nn_W1'], p['nn_b1'].reshape(1, NF),
      p['nn_W2'], p['nn_b2'].reshape(1, NF),
      p['lin2_W'], p['lin2_b'].reshape(1, NF),
      p['cls_W1'], p['cls_b1'].reshape(1, NF),
      p['cls_W2'], p['cls_b2'].reshape(1, NCLS),
      p['prp_W1'], p['prp_b1'].reshape(1, NF),
      p['prp_W2'], p['prp_b2'].reshape(1, NIND))


def kernel(pos_query, pos_ctx, node_attr_ctx, batch_query, batch_ctx, params):
    pq8 = jnp.zeros((NQ, 8), jnp.float32).at[:, :3].set(pos_query)
    pcT8 = jnp.zeros((8, NC), jnp.float32).at[:3, :].set(pos_ctx.T)
    bq = batch_query.reshape(NQ, 1)
    bc = batch_ctx.reshape(1, NC)

    b_lo = batch_query.reshape(NBLK, QB)[:, 0]
    b_hi = batch_query.reshape(NBLK, QB)[:, -1]
    start = jnp.searchsorted(batch_ctx, b_lo, side='left').astype(jnp.int32)
    end = jnp.searchsorted(batch_ctx, b_hi, side='right').astype(jnp.int32)
    t0s = start // TW
    t1s = jnp.maximum((end + TW - 1) // TW, t0s + 1)

    A = _precompute_a(node_attr_ctx, params['lin1_W'])
    idx, d2 = _knn(pq8, pcT8, bq, bc, t0s, t1s)
    G = _sc_gather_fn()(A, idx.reshape(NE))
    y_cls, y_ind = _edge_mlp(G, d2, params)
    return (y_cls, y_ind)
